# baseline (speedup 1.0000x reference)
import functools

import jax
import jax.numpy as jnp
from jax import lax
from jax.experimental import pallas as pl
from jax.experimental.pallas import tpu as pltpu
from jax.experimental.pallas import tpu_sc as plsc

NNODES = 10000
HDIM = 128
NGRP = 16
EDGES = 320000

NC = 2
NS = 16
CH = 128
NPAD = 10240
NCH = 80
EPW = NCH * CH
EPAD = NC * NS * EPW
WB = NPAD // NS

MB = 1024
RB = 1000
NB = NNODES // RB

@functools.cache
def _get_sc_agg():
    mesh = plsc.VectorSubcoreMesh(core_axis_name="c", subcore_axis_name="s")

    @functools.partial(
        pl.kernel,
        mesh=mesh,
        out_type=jax.ShapeDtypeStruct((NC, NPAD, HDIM), jnp.float32),
        scratch_types=[
            pltpu.VMEM((NCH // 2, CH), jnp.int32),
            pltpu.VMEM((NCH // 2, CH), jnp.int32),
            pltpu.VMEM((CH, HDIM), jnp.float32),
            pltpu.VMEM((CH, HDIM), jnp.float32),
            pltpu.VMEM_SHARED((NPAD, HDIM), jnp.float32),
            pltpu.SemaphoreType.DMA,
            pltpu.SemaphoreType.DMA,
            pltpu.SemaphoreType.DMA,
            pltpu.SemaphoreType.DMA,
        ],
    )
    def _sc_agg(x_hbm, srcw_hbm, dstw_hbm, zero_hbm, out_hbm,
                src_all, dst_all, rows_a, rows_b, acc_sh,
                sem_a, sem_b, sem_sa, sem_sb):
        c = lax.axis_index("c")
        s = lax.axis_index("s")
        w = c * NS + s

        pltpu.sync_copy(zero_hbm.at[pl.ds(s * WB, WB), :],
                        acc_sh.at[pl.ds(s * WB, WB), :])

        plsc.subcore_barrier()

        nchp = NCH // 2
        for ph in range(2):
            pltpu.sync_copy(srcw_hbm.at[w, pl.ds(ph * nchp, nchp)], src_all)
            pltpu.sync_copy(dstw_hbm.at[w, pl.ds(ph * nchp, nchp)], dst_all)
            pltpu.async_copy(x_hbm.at[src_all.at[0]], rows_a, sem_a)
            pltpu.async_copy(x_hbm.at[src_all.at[1]], rows_b, sem_b)

            @pl.loop(0, nchp // 2)
            def _(j):
                k0 = 2 * j
                pltpu.make_async_copy(x_hbm.at[src_all.at[k0]],
                                      rows_a, sem_a).wait()
                h_sa = pltpu.async_copy(rows_a, acc_sh.at[dst_all.at[k0]],
                                        sem_sa, add=True)
                pltpu.make_async_copy(x_hbm.at[src_all.at[k0 + 1]],
                                      rows_b, sem_b).wait()
                h_sb = pltpu.async_copy(rows_b, acc_sh.at[dst_all.at[k0 + 1]],
                                        sem_sb, add=True)
                h_sa.wait()

                @pl.when(j < nchp // 2 - 1)
                def _():
                    pltpu.async_copy(x_hbm.at[src_all.at[k0 + 2]],
                                     rows_a, sem_a)

                h_sb.wait()

                @pl.when(j < nchp // 2 - 1)
                def _():
                    pltpu.async_copy(x_hbm.at[src_all.at[k0 + 3]],
                                     rows_b, sem_b)

        plsc.subcore_barrier()
        pltpu.sync_copy(acc_sh.at[pl.ds(s * WB, WB), :],
                        out_hbm.at[c, pl.ds(s * WB, WB), :])

    return _sc_agg


def _dot(a, b):
    return lax.dot_general(a, b, (((1,), (0,)), ((), ())),
                           precision=lax.Precision.HIGHEST,
                           preferred_element_type=jnp.float32)


def _mlp_body(x_ref, a0_ref, a1_ref, w1_ref, b1_ref, w2_ref, b2_ref, o_ref):
    h = x_ref[...] + a0_ref[0] + a1_ref[0]
    z = jnp.maximum(_dot(h, w1_ref[...]) + b1_ref[...], 0.0)
    o_ref[...] = jnp.maximum(_dot(z, w2_ref[...]) + b2_ref[...], 0.0)


def _mlp(x, agg, w1, b1, w2, b2):
    return pl.pallas_call(
        _mlp_body,
        grid=(NPAD // MB,),
        in_specs=[
            pl.BlockSpec((MB, HDIM), lambda i: (i, 0)),
            pl.BlockSpec((1, MB, HDIM), lambda i: (0, i, 0)),
            pl.BlockSpec((1, MB, HDIM), lambda i: (1, i, 0)),
            pl.BlockSpec((HDIM, HDIM), lambda i: (0, 0)),
            pl.BlockSpec((1, HDIM), lambda i: (0, 0)),
            pl.BlockSpec((HDIM, HDIM), lambda i: (0, 0)),
            pl.BlockSpec((1, HDIM), lambda i: (0, 0)),
        ],
        out_specs=pl.BlockSpec((MB, HDIM), lambda i: (i, 0)),
        out_shape=jax.ShapeDtypeStruct((NPAD, HDIM), jnp.float32),
    )(x, agg, agg, w1, b1.reshape(1, HDIM), w2, b2.reshape(1, HDIM))


def _pool_body(x_ref, b_ref, gw1_ref, gb1_ref, gw2_ref, gb2_ref,
               pw_ref, pb_ref, o_ref, m_scr, d_scr, s_scr):
    i = pl.program_id(0)

    @pl.when(i == 0)
    def _():
        m_scr[...] = jnp.full((NGRP, 1), -3e38, jnp.float32)
        d_scr[...] = jnp.zeros((NGRP, 1), jnp.float32)
        s_scr[...] = jnp.zeros((NGRP, HDIM), jnp.float32)

    def gate_and_onehot():
        xb = x_ref[...]
        z = jnp.maximum(_dot(xb, gw1_ref[...]) + gb1_ref[...], 0.0)
        gate = _dot(z, gw2_ref[...]) + gb2_ref[...]
        onehot = (b_ref[...] == lax.broadcasted_iota(
            jnp.int32, (RB, NGRP), 1)).astype(jnp.float32)
        return gate, onehot

    @pl.when(i < NB)
    def _():
        gate, onehot = gate_and_onehot()
        masked = jnp.where(onehot > 0.0, gate, -3e38)
        m_scr[...] = jnp.maximum(m_scr[...], jnp.max(masked, axis=0)[:, None])

    @pl.when(jnp.logical_and(i >= NB, i < 2 * NB))
    def _():
        gate, onehot = gate_and_onehot()
        m_rows = _dot(onehot, m_scr[...])
        e = jnp.exp(gate - m_rows)
        ot = lax.dot_general(onehot, e, (((0,), (0,)), ((), ())),
                             precision=lax.Precision.HIGHEST,
                             preferred_element_type=jnp.float32)
        d_scr[...] += ot
        s_scr[...] += lax.dot_general(
            onehot, e * x_ref[...], (((0,), (0,)), ((), ())),
            precision=lax.Precision.HIGHEST,
            preferred_element_type=jnp.float32)

    @pl.when(i == 2 * NB)
    def _():
        pooled = s_scr[...] / (d_scr[...] + 1e-16)
        o_ref[...] = _dot(pooled, pw_ref[...]) + pb_ref[...]


def _pool_row_map(i):
    r = jnp.where(i < NB, i, jnp.where(i < 2 * NB, i - NB, 0))
    return (r, 0)


def _pool(x, batch2, gw1, gb1, gw2, gb2, pw, pb):
    return pl.pallas_call(
        _pool_body,
        grid=(2 * NB + 1,),
        in_specs=[
            pl.BlockSpec((RB, HDIM), _pool_row_map),
            pl.BlockSpec((RB, 1), _pool_row_map),
            pl.BlockSpec((HDIM, HDIM), lambda i: (0, 0)),
            pl.BlockSpec((1, HDIM), lambda i: (0, 0)),
            pl.BlockSpec((HDIM, 1), lambda i: (0, 0)),
            pl.BlockSpec((1, 1), lambda i: (0, 0)),
            pl.BlockSpec((HDIM, pw.shape[1]), lambda i: (0, 0)),
            pl.BlockSpec((1, pw.shape[1]), lambda i: (0, 0)),
        ],
        out_specs=pl.BlockSpec((NGRP, pw.shape[1]), lambda i: (0, 0)),
        out_shape=jax.ShapeDtypeStruct((NGRP, pw.shape[1]), jnp.float32),
        scratch_shapes=[
            pltpu.VMEM((NGRP, 1), jnp.float32),
            pltpu.VMEM((NGRP, 1), jnp.float32),
            pltpu.VMEM((NGRP, HDIM), jnp.float32),
        ],
    )(x, batch2, gw1, gb1.reshape(1, HDIM), gw2, gb2.reshape(1, 1),
      pw, pb.reshape(1, pw.shape[1]))


def kernel(x, edge_index, batch,
           c0_W1, c0_b1, c0_W2, c0_b2,
           c1_W1, c1_b1, c1_W2, c1_b2,
           c2_W1, c2_b1, c2_W2, c2_b2,
           g_W1, g_b1, g_W2, g_b2, p_W, p_b):
    src = edge_index[0]
    dst = edge_index[1]
    npad_e = EPAD - EDGES
    fill = jnp.arange(npad_e, dtype=jnp.int32)
    src_p = jnp.concatenate([src, fill % NNODES])
    dst_p = jnp.concatenate([dst, NNODES + (fill % (NPAD - NNODES))])
    srcw = src_p.reshape(NC * NS, NCH, CH)
    dstw = dst_p.reshape(NC * NS, NCH, CH)
    zeros = jnp.zeros((NPAD, HDIM), jnp.float32)

    xp = jnp.pad(x, ((0, NPAD - NNODES), (0, 0)))
    for w1, b1, w2, b2 in ((c0_W1, c0_b1, c0_W2, c0_b2),
                           (c1_W1, c1_b1, c1_W2, c1_b2),
                           (c2_W1, c2_b1, c2_W2, c2_b2)):
        agg = _get_sc_agg()(xp, srcw, dstw, zeros)
        xp = _mlp(xp, agg, w1, b1, w2, b2)

    return _pool(xp, batch[:, None], g_W1, g_b1, g_W2, g_b2, p_W, p_b)


# speedup vs baseline: 1.2387x; 1.2387x over previous
import functools

import jax
import jax.numpy as jnp
from jax import lax
from jax.experimental import pallas as pl
from jax.experimental.pallas import tpu as pltpu
from jax.experimental.pallas import tpu_sc as plsc

NNODES = 10000
HDIM = 128
NGRP = 16
EDGES = 320000

NC = 2
NS = 16
CH = 128
NPAD = 10240
NCH = 80
EPW = NCH * CH
EPAD = NC * NS * EPW
WB = NPAD // NS

MB = 1024
RB = 1000
NB = NNODES // RB

@functools.cache
def _get_sc_agg():
    mesh = plsc.VectorSubcoreMesh(core_axis_name="c", subcore_axis_name="s")

    @functools.partial(
        pl.kernel,
        mesh=mesh,
        out_type=jax.ShapeDtypeStruct((NC, NPAD, HDIM), jnp.float32),
        scratch_types=[
            pltpu.VMEM((NCH // 2, CH), jnp.int32),
            pltpu.VMEM((NCH // 2, CH), jnp.int32),
            pltpu.VMEM((CH, HDIM), jnp.float32),
            pltpu.VMEM((CH, HDIM), jnp.float32),
            pltpu.VMEM_SHARED((NPAD, HDIM), jnp.float32),
            pltpu.SemaphoreType.DMA,
            pltpu.SemaphoreType.DMA,
        ],
    )
    def _sc_agg(x_hbm, srcw_hbm, dstw_hbm, zero_hbm, out_hbm,
                src_all, dst_all, rows_a, rows_b, acc_sh, sem_a, sem_b):
        c = lax.axis_index("c")
        s = lax.axis_index("s")
        w = c * NS + s

        pltpu.sync_copy(zero_hbm.at[pl.ds(s * WB, WB), :],
                        acc_sh.at[pl.ds(s * WB, WB), :])

        plsc.subcore_barrier()

        nchp = NCH // 2
        for ph in range(2):
            pltpu.sync_copy(srcw_hbm.at[w, pl.ds(ph * nchp, nchp)], src_all)
            pltpu.sync_copy(dstw_hbm.at[w, pl.ds(ph * nchp, nchp)], dst_all)
            pltpu.async_copy(x_hbm.at[src_all.at[0]], rows_a, sem_a)

            @pl.loop(0, nchp // 2)
            def _(j):
                k0 = 2 * j
                pltpu.async_copy(x_hbm.at[src_all.at[k0 + 1]], rows_b, sem_b)
                pltpu.make_async_copy(x_hbm.at[src_all.at[k0]],
                                      rows_a, sem_a).wait()
                pltpu.sync_copy(rows_a, acc_sh.at[dst_all.at[k0]], add=True)

                @pl.when(j < nchp // 2 - 1)
                def _():
                    pltpu.async_copy(x_hbm.at[src_all.at[k0 + 2]],
                                     rows_a, sem_a)

                pltpu.make_async_copy(x_hbm.at[src_all.at[k0 + 1]],
                                      rows_b, sem_b).wait()
                pltpu.sync_copy(rows_b, acc_sh.at[dst_all.at[k0 + 1]],
                                add=True)

        plsc.subcore_barrier()
        pltpu.sync_copy(acc_sh.at[pl.ds(s * WB, WB), :],
                        out_hbm.at[c, pl.ds(s * WB, WB), :])

    return _sc_agg


def _dot(a, b):
    return lax.dot_general(a, b, (((1,), (0,)), ((), ())),
                           precision=lax.Precision.HIGHEST,
                           preferred_element_type=jnp.float32)


def _mlp_body(x_ref, a0_ref, a1_ref, w1_ref, b1_ref, w2_ref, b2_ref, o_ref):
    h = x_ref[...] + a0_ref[0] + a1_ref[0]
    z = jnp.maximum(_dot(h, w1_ref[...]) + b1_ref[...], 0.0)
    o_ref[...] = jnp.maximum(_dot(z, w2_ref[...]) + b2_ref[...], 0.0)


def _mlp(x, agg, w1, b1, w2, b2):
    return pl.pallas_call(
        _mlp_body,
        grid=(NPAD // MB,),
        in_specs=[
            pl.BlockSpec((MB, HDIM), lambda i: (i, 0)),
            pl.BlockSpec((1, MB, HDIM), lambda i: (0, i, 0)),
            pl.BlockSpec((1, MB, HDIM), lambda i: (1, i, 0)),
            pl.BlockSpec((HDIM, HDIM), lambda i: (0, 0)),
            pl.BlockSpec((1, HDIM), lambda i: (0, 0)),
            pl.BlockSpec((HDIM, HDIM), lambda i: (0, 0)),
            pl.BlockSpec((1, HDIM), lambda i: (0, 0)),
        ],
        out_specs=pl.BlockSpec((MB, HDIM), lambda i: (i, 0)),
        out_shape=jax.ShapeDtypeStruct((NPAD, HDIM), jnp.float32),
    )(x, agg, agg, w1, b1.reshape(1, HDIM), w2, b2.reshape(1, HDIM))


def _mlp_pool_body(x_ref, a0_ref, a1_ref, w1_ref, b1_ref, w2_ref, b2_ref,
                   b_ref, gw1_ref, gb1_ref, gw2_ref, gb2_ref,
                   pw_ref, pb_ref, o_ref, m_scr, d_scr, s_scr):
    i = pl.program_id(0)

    @pl.when(i == 0)
    def _():
        m_scr[...] = jnp.full((NGRP, 1), -3e38, jnp.float32)
        d_scr[...] = jnp.zeros((NGRP, 1), jnp.float32)
        s_scr[...] = jnp.zeros((NGRP, HDIM), jnp.float32)

    h = x_ref[...] + a0_ref[0] + a1_ref[0]
    z = jnp.maximum(_dot(h, w1_ref[...]) + b1_ref[...], 0.0)
    x3 = jnp.maximum(_dot(z, w2_ref[...]) + b2_ref[...], 0.0)

    zg = jnp.maximum(_dot(x3, gw1_ref[...]) + gb1_ref[...], 0.0)
    gate = _dot(zg, gw2_ref[...]) + gb2_ref[...]
    onehot = (b_ref[...] == lax.broadcasted_iota(
        jnp.int32, (RB, NGRP), 1)).astype(jnp.float32)

    masked = jnp.where(onehot > 0.0, gate, -3e38)
    m_new = jnp.maximum(m_scr[...], jnp.max(masked, axis=0)[:, None])
    alpha = jnp.exp(m_scr[...] - m_new)
    m_rows = _dot(onehot, m_new)
    e = jnp.exp(gate - m_rows)
    d_scr[...] = d_scr[...] * alpha + lax.dot_general(
        onehot, e, (((0,), (0,)), ((), ())),
        precision=lax.Precision.HIGHEST,
        preferred_element_type=jnp.float32)
    s_scr[...] = s_scr[...] * alpha + lax.dot_general(
        onehot, e * x3, (((0,), (0,)), ((), ())),
        precision=lax.Precision.HIGHEST,
        preferred_element_type=jnp.float32)
    m_scr[...] = m_new

    @pl.when(i == NB - 1)
    def _():
        pooled = s_scr[...] / (d_scr[...] + 1e-16)
        o_ref[...] = _dot(pooled, pw_ref[...]) + pb_ref[...]


def _mlp_pool(x, agg, w1, b1, w2, b2, batch2, gw1, gb1, gw2, gb2, pw, pb):
    return pl.pallas_call(
        _mlp_pool_body,
        grid=(NB,),
        in_specs=[
            pl.BlockSpec((RB, HDIM), lambda i: (i, 0)),
            pl.BlockSpec((1, RB, HDIM), lambda i: (0, i, 0)),
            pl.BlockSpec((1, RB, HDIM), lambda i: (1, i, 0)),
            pl.BlockSpec((HDIM, HDIM), lambda i: (0, 0)),
            pl.BlockSpec((1, HDIM), lambda i: (0, 0)),
            pl.BlockSpec((HDIM, HDIM), lambda i: (0, 0)),
            pl.BlockSpec((1, HDIM), lambda i: (0, 0)),
            pl.BlockSpec((RB, 1), lambda i: (i, 0)),
            pl.BlockSpec((HDIM, HDIM), lambda i: (0, 0)),
            pl.BlockSpec((1, HDIM), lambda i: (0, 0)),
            pl.BlockSpec((HDIM, 1), lambda i: (0, 0)),
            pl.BlockSpec((1, 1), lambda i: (0, 0)),
            pl.BlockSpec((HDIM, pw.shape[1]), lambda i: (0, 0)),
            pl.BlockSpec((1, pw.shape[1]), lambda i: (0, 0)),
        ],
        out_specs=pl.BlockSpec((NGRP, pw.shape[1]), lambda i: (0, 0)),
        out_shape=jax.ShapeDtypeStruct((NGRP, pw.shape[1]), jnp.float32),
        scratch_shapes=[
            pltpu.VMEM((NGRP, 1), jnp.float32),
            pltpu.VMEM((NGRP, 1), jnp.float32),
            pltpu.VMEM((NGRP, HDIM), jnp.float32),
        ],
    )(x, agg, agg, w1, b1.reshape(1, HDIM), w2, b2.reshape(1, HDIM),
      batch2, gw1, gb1.reshape(1, HDIM), gw2, gb2.reshape(1, 1),
      pw, pb.reshape(1, pw.shape[1]))


def kernel(x, edge_index, batch,
           c0_W1, c0_b1, c0_W2, c0_b2,
           c1_W1, c1_b1, c1_W2, c1_b2,
           c2_W1, c2_b1, c2_W2, c2_b2,
           g_W1, g_b1, g_W2, g_b2, p_W, p_b):
    src = edge_index[0]
    dst = edge_index[1]
    npad_e = EPAD - EDGES
    fill = jnp.arange(npad_e, dtype=jnp.int32)
    src_p = jnp.concatenate([src, fill % NNODES])
    dst_p = jnp.concatenate([dst, NNODES + (fill % (NPAD - NNODES))])
    srcw = src_p.reshape(NC * NS, NCH, CH)
    dstw = dst_p.reshape(NC * NS, NCH, CH)
    zeros = jnp.zeros((NPAD, HDIM), jnp.float32)

    xp = jnp.pad(x, ((0, NPAD - NNODES), (0, 0)))
    for w1, b1, w2, b2 in ((c0_W1, c0_b1, c0_W2, c0_b2),
                           (c1_W1, c1_b1, c1_W2, c1_b2)):
        agg = _get_sc_agg()(xp, srcw, dstw, zeros)
        xp = _mlp(xp, agg, w1, b1, w2, b2)

    agg = _get_sc_agg()(xp, srcw, dstw, zeros)
    return _mlp_pool(xp, agg, c2_W1, c2_b1, c2_W2, c2_b2, batch[:, None],
                     g_W1, g_b1, g_W2, g_b2, p_W, p_b)
